# baseline (speedup 1.0000x reference)
import functools

import numpy as np

import jax
import jax.numpy as jnp
from jax import lax
from jax.experimental import pallas as pl
from jax.experimental.pallas import tpu as pltpu

N = 10000
E = 320000
LANES = 128
R = 4096
K_HOMO = int(E * 0.8)
K_HET = int(E * 0.2)


def _make_stage_tables(n):
    kks, js = [], []
    kk = 2
    while kk <= n:
        j = kk >> 1
        while j >= 1:
            kks.append(kk)
            js.append(j)
            j >>= 1
        kk <<= 1
    return np.array(kks, np.int32), np.array(js, np.int32)


_KK_TAB, _J_TAB = _make_stage_tables(R * LANES)
_NSTAGES = len(_KK_TAB)


def _sort_kernel(kk_ref, j_ref, val_ref, oidx1_ref, oidx2_ref, kscr, iscr):
    row = lax.broadcasted_iota(jnp.int32, (R, LANES), 0)
    lane = lax.broadcasted_iota(jnp.int32, (R, LANES), 1)
    i = row * LANES + lane

    def do_sort(key, idx):
        def body(t, carry):
            key, idx = carry
            kk = kk_ref[t]
            j = j_ref[t]

            def lane_fn(key, idx):
                a_k = pltpu.roll(key, LANES - j, axis=1)
                b_k = pltpu.roll(key, j, axis=1)
                a_i = pltpu.roll(idx, LANES - j, axis=1)
                b_i = pltpu.roll(idx, j, axis=1)
                sel = (lane & j) == 0
                return jnp.where(sel, a_k, b_k), jnp.where(sel, a_i, b_i)

            def row_fn(key, idx):
                jr = j >> 7
                kscr[pl.ds(0, R), :] = key
                kscr[pl.ds(R, R), :] = key
                iscr[pl.ds(0, R), :] = idx
                iscr[pl.ds(R, R), :] = idx
                a_k = kscr[pl.ds(jr, R), :]
                b_k = kscr[pl.ds(R - jr, R), :]
                a_i = iscr[pl.ds(jr, R), :]
                b_i = iscr[pl.ds(R - jr, R), :]
                sel = (row & jr) == 0
                return jnp.where(sel, a_k, b_k), jnp.where(sel, a_i, b_i)

            pk, pi = lax.cond(j < LANES, lane_fn, row_fn, key, idx)
            less = (key < pk) | ((key == pk) & (idx < pi))
            lower = (i & j) == 0
            up = (i & kk) == 0
            take_a = (lower == up) == less
            key = jnp.where(take_a, key, pk)
            idx = jnp.where(take_a, idx, pi)
            return key, idx

        return lax.fori_loop(0, _NSTAGES, body, (key, idx))

    v = val_ref[...]
    b = lax.bitcast_convert_type(v, jnp.uint32)
    mask = jnp.where(b >= jnp.uint32(0x80000000), jnp.uint32(0xFFFFFFFF),
                     jnp.uint32(0x80000000))
    u = b ^ mask
    pad = i >= E
    idx0 = jnp.where(pad, jnp.int32(0x7FFFFFFF), i)
    _, i1 = do_sort(jnp.where(pad, jnp.uint32(0xFFFFFFFF), ~u), idx0)
    oidx1_ref[...] = i1
    _, i2 = do_sort(jnp.where(pad, jnp.uint32(0xFFFFFFFF), u), idx0)
    oidx2_ref[...] = i2


_sort_call = pl.pallas_call(
    _sort_kernel,
    in_specs=[
        pl.BlockSpec(memory_space=pltpu.SMEM),
        pl.BlockSpec(memory_space=pltpu.SMEM),
        pl.BlockSpec(memory_space=pltpu.VMEM),
    ],
    out_specs=[
        pl.BlockSpec(memory_space=pltpu.VMEM),
        pl.BlockSpec(memory_space=pltpu.VMEM),
    ],
    out_shape=[
        jax.ShapeDtypeStruct((R, LANES), jnp.int32),
        jax.ShapeDtypeStruct((R, LANES), jnp.int32),
    ],
    scratch_shapes=[
        pltpu.VMEM((2 * R, LANES), jnp.uint32),
        pltpu.VMEM((2 * R, LANES), jnp.int32),
    ],
)


def _gcn_conv(x, edge_index, W, b):
    h = x @ W
    loop = jnp.arange(N, dtype=edge_index.dtype)
    src = jnp.concatenate([edge_index[0], loop])
    dst = jnp.concatenate([edge_index[1], loop])
    deg = jnp.zeros((N,), dtype=h.dtype).at[dst].add(jnp.ones(src.shape[0], dtype=h.dtype))
    dinv = jnp.where(deg > 0, 1.0 / jnp.sqrt(deg), 0.0)
    norm = dinv[src] * dinv[dst]
    msg = h[src] * norm[:, None]
    out = jnp.zeros((N, h.shape[1]), dtype=h.dtype).at[dst].add(msg)
    return out + b


def kernel(x, edge_index, W1, b1, W2, b2):
    xM1 = jax.nn.relu(_gcn_conv(x, edge_index, W1, b1))
    xM2 = _gcn_conv(xM1, edge_index, W2, b2)
    value = (xM2[edge_index[0]] * xM2[edge_index[1]]).sum(axis=1)
    vp = jnp.zeros((R * LANES,), jnp.float32).at[:E].set(value).reshape(R, LANES)
    i1, i2 = _sort_call(jnp.asarray(_KK_TAB), jnp.asarray(_J_TAB), vp)
    homo = i1.reshape(-1)[:K_HOMO]
    het = i2.reshape(-1)[:K_HET]
    return (jnp.take(edge_index, homo, axis=1), jnp.take(edge_index, het, axis=1))


# speedup vs baseline: 1.3613x; 1.3613x over previous
import functools

import jax
import jax.numpy as jnp
from jax import lax
from jax.experimental import pallas as pl
from jax.experimental.pallas import tpu as pltpu
from jax.experimental.pallas import tpu_sc as plsc

N = 10000
E = 320000
OUT_DIM = 16
LANES = 128
R = 4096
PADR = 2048
K_HOMO = int(E * 0.8)
K_HET = int(E * 0.2)


def _sort_kernel(val_ref, oidx_ref, otgt_ref, kscr, iscr):
    n = R * LANES
    row = lax.broadcasted_iota(jnp.int32, (R, LANES), 0)
    lane = lax.broadcasted_iota(jnp.int32, (R, LANES), 1)
    i = row * LANES + lane

    def row_partner(x, jr, scr):
        scr[pl.ds(PADR, R), :] = x
        a = scr[pl.ds(PADR + jr, R), :]
        b = scr[pl.ds(PADR - jr, R), :]
        return a, b

    def cmpx(key, idx, pk, pi, jbit, kk, active=None):
        less = (key < pk) | ((key == pk) & (idx < pi))
        take_a = (((i & jbit) == 0) == ((i & kk) == 0)) == less
        if active is not None:
            take_a = take_a | (~active)
        return jnp.where(take_a, key, pk), jnp.where(take_a, idx, pi)

    def phase(m, carry):
        key, idx = carry
        kk = jnp.int32(1) << m

        def row_stage(t, carry):
            key, idx = carry
            jr = (kk >> 8) >> t
            a_k, b_k = row_partner(key, jr, kscr)
            a_i, b_i = row_partner(idx, jr, iscr)
            sel = (row & jr) == 0
            pk = jnp.where(sel, a_k, b_k)
            pi = jnp.where(sel, a_i, b_i)
            return cmpx(key, idx, pk, pi, jr << 7, kk)

        key, idx = lax.fori_loop(0, jnp.maximum(m - 7, 0), row_stage,
                                 (key, idx))
        for s in (64, 32, 16, 8, 4, 2, 1):
            a_k = pltpu.roll(key, LANES - s, axis=1)
            b_k = pltpu.roll(key, s, axis=1)
            a_i = pltpu.roll(idx, LANES - s, axis=1)
            b_i = pltpu.roll(idx, s, axis=1)
            sel = (lane & s) == 0
            pk = jnp.where(sel, a_k, b_k)
            pi = jnp.where(sel, a_i, b_i)
            active = jnp.broadcast_to(jnp.int32(s) <= (kk >> 1), key.shape)
            key, idx = cmpx(key, idx, pk, pi, s, kk, active)
        return key, idx

    v = val_ref[...]
    b = lax.bitcast_convert_type(v, jnp.uint32)
    mask = jnp.where(b >= jnp.uint32(0x80000000), jnp.uint32(0xFFFFFFFF),
                     jnp.uint32(0x80000000))
    u = b ^ mask
    pad = i >= E
    key0 = jnp.where(pad, jnp.uint32(0xFFFFFFFF), u)
    idx0 = jnp.where(pad, jnp.int32(0x7FFFFFFF), i)
    key_s, idx_s = lax.fori_loop(1, 20, phase, (key0, idx0))
    oidx_ref[...] = idx_s

    def row_shift_down(x, q, fill):
        iscr[pl.ds(PADR, R), :] = x
        y = iscr[pl.ds(PADR - q, R), :]
        return jnp.where(row >= q, y, fill)

    def row_shift_up(x, q, fill):
        iscr[pl.ds(PADR, R), :] = x
        y = iscr[pl.ds(PADR + q, R), :]
        return jnp.where(row < R - q, y, fill)

    kscr[pl.ds(PADR, R), :] = key_s
    km1 = kscr[pl.ds(PADR - 1, R), :]
    prev = jnp.where(lane >= 1, pltpu.roll(key_s, 1, axis=1),
                     pltpu.roll(km1, 1, axis=1))
    bdry = (key_s != prev) | (i == 0)
    kp1 = kscr[pl.ds(PADR + 1, R), :]
    nxt = jnp.where(lane < LANES - 1, pltpu.roll(key_s, LANES - 1, axis=1),
                    pltpu.roll(kp1, LANES - 1, axis=1))
    endb = (key_s != nxt) | (i == n - 1)

    p = jnp.where(bdry, i, 0)
    for k in (1, 2, 4, 8, 16, 32, 64):
        sh = pltpu.roll(p, k, axis=1)
        p = jnp.maximum(p, jnp.where(lane >= k, sh, 0))
    t_row = jnp.broadcast_to(jnp.max(p, axis=1, keepdims=True), p.shape)
    x = row_shift_down(t_row, 1, 0)

    def scan_fwd(t, x):
        q = jnp.int32(1) << t
        return jnp.maximum(x, row_shift_down(x, q, 0))

    x = lax.fori_loop(0, 12, scan_fwd, x)
    s_run = jnp.maximum(p, x)

    big = jnp.int32(0x7FFFFFFF)
    p2 = jnp.where(endb, i, big)
    for k in (1, 2, 4, 8, 16, 32, 64):
        sh = pltpu.roll(p2, LANES - k, axis=1)
        p2 = jnp.minimum(p2, jnp.where(lane < LANES - k, sh, big))
    u_row = jnp.broadcast_to(jnp.min(p2, axis=1, keepdims=True), p2.shape)
    y = row_shift_up(u_row, 1, big)

    def scan_bwd(t, y):
        q = jnp.int32(1) << t
        return jnp.minimum(y, row_shift_up(y, q, big))

    y = lax.fori_loop(0, 12, scan_bwd, y)
    e_run = jnp.minimum(p2, y)
    otgt_ref[...] = s_run + e_run - i


_sort_call = pl.pallas_call(
    _sort_kernel,
    in_specs=[
        pl.BlockSpec(memory_space=pltpu.VMEM),
    ],
    out_specs=[
        pl.BlockSpec(memory_space=pltpu.VMEM),
        pl.BlockSpec(memory_space=pltpu.VMEM),
    ],
    out_shape=[
        jax.ShapeDtypeStruct((R, LANES), jnp.int32),
        jax.ShapeDtypeStruct((R, LANES), jnp.int32),
    ],
    scratch_shapes=[
        pltpu.VMEM((R + 2 * PADR, LANES), jnp.uint32),
        pltpu.VMEM((R + 2 * PADR, LANES), jnp.int32),
    ],
)


EFULL = E + N
_NW = 32
_CH = 120
_BPAD = 330240
_EPAD = 322560


def _make_row_gather(BPAD):
    D = 128
    mesh = plsc.VectorSubcoreMesh(core_axis_name="c", subcore_axis_name="s")
    cpw = (BPAD // _NW) // _CH

    @functools.partial(
        pl.kernel,
        mesh=mesh,
        out_type=jax.ShapeDtypeStruct((BPAD, D), jnp.float32),
        scratch_types=[
            pltpu.VMEM((_CH,), jnp.int32),
            pltpu.VMEM((_CH, D), jnp.float32),
            pltpu.SemaphoreType.DMA,
        ],
    )
    def row_gather(table_hbm, idx_hbm, out_hbm, iv, rv, sem):
        wid = lax.axis_index("s") * 2 + lax.axis_index("c")

        def chunk(c, carry):
            off = wid * (cpw * _CH) + c * _CH
            pltpu.sync_copy(idx_hbm.at[pl.ds(off, _CH)], iv)
            pltpu.async_copy(table_hbm.at[iv], rv, sem).wait()
            pltpu.sync_copy(rv, out_hbm.at[pl.ds(off, _CH)])
            return carry

        lax.fori_loop(0, cpw, chunk, 0)

    return row_gather


def _gcn_conv(x, edge_index, W, b, gather_fn):
    h = x @ W
    loop = jnp.arange(N, dtype=edge_index.dtype)
    src = jnp.concatenate([edge_index[0], loop])
    dst = jnp.concatenate([edge_index[1], loop])
    deg = jnp.zeros((N,), dtype=h.dtype).at[dst].add(jnp.ones(src.shape[0], dtype=h.dtype))
    dinv = jnp.where(deg > 0, 1.0 / jnp.sqrt(deg), 0.0)
    norm = dinv[src] * dinv[dst]
    src_pad = jnp.concatenate(
        [src, jnp.zeros((_BPAD - EFULL,), dtype=src.dtype)])
    h128 = jnp.concatenate(
        [h, jnp.zeros((N, 128 - h.shape[1]), dtype=h.dtype)], axis=1)
    hsrc = gather_fn(h128, src_pad)[:EFULL, :h.shape[1]]
    msg = hsrc * norm[:, None]
    out = jnp.zeros((N, h.shape[1]), dtype=h.dtype).at[dst].add(msg)
    return out + b


_gather_full = _make_row_gather(_BPAD)
_gather_edge = _make_row_gather(_EPAD)


def kernel(x, edge_index, W1, b1, W2, b2):
    xM1 = jax.nn.relu(_gcn_conv(x, edge_index, W1, b1, _gather_full))
    xM2 = _gcn_conv(xM1, edge_index, W2, b2, _gather_full)
    zpad = jnp.zeros((_EPAD - E,), dtype=edge_index.dtype)
    xM2p = jnp.concatenate(
        [xM2, jnp.zeros((N, 128 - OUT_DIM), dtype=xM2.dtype)], axis=1)
    g0 = _gather_edge(xM2p, jnp.concatenate([edge_index[0], zpad]))[:E, :OUT_DIM]
    g1 = _gather_edge(xM2p, jnp.concatenate([edge_index[1], zpad]))[:E, :OUT_DIM]
    value = (g0 * g1).sum(axis=1)
    vp = jnp.zeros((R * LANES,), jnp.float32).at[:E].set(value).reshape(R, LANES)
    oidx, otgt = _sort_call(vp)
    idx_s = oidx.reshape(-1)
    tgt = otgt.reshape(-1)
    het = idx_s[:K_HET]
    homo = jnp.take(idx_s, tgt[E - 1 - jnp.arange(K_HOMO)])
    return (jnp.take(edge_index, homo, axis=1), jnp.take(edge_index, het, axis=1))


# speedup vs baseline: 2.1786x; 1.6004x over previous
import functools

import jax
import jax.numpy as jnp
from jax import lax
from jax.experimental import pallas as pl
from jax.experimental.pallas import tpu as pltpu
from jax.experimental.pallas import tpu_sc as plsc

N = 10000
E = 320000
OUT_DIM = 16
LANES = 128
R = 4096
PADR = 2048
K_HOMO = int(E * 0.8)
K_HET = int(E * 0.2)


def _sort_kernel(val_ref, oidx_ref, otgt_ref, kscr, iscr):
    n = R * LANES
    row = lax.broadcasted_iota(jnp.int32, (R, LANES), 0)
    lane = lax.broadcasted_iota(jnp.int32, (R, LANES), 1)
    i = row * LANES + lane

    def row_partner(x, jr, scr):
        scr[pl.ds(PADR, R), :] = x
        a = scr[pl.ds(PADR + jr, R), :]
        b = scr[pl.ds(PADR - jr, R), :]
        return a, b

    def cmpx(key, idx, pk, pi, jbit, kk, active=None):
        less = (key < pk) | ((key == pk) & (idx < pi))
        take_a = (((i & jbit) == 0) == ((i & kk) == 0)) == less
        if active is not None:
            take_a = take_a | (~active)
        return jnp.where(take_a, key, pk), jnp.where(take_a, idx, pi)

    def phase(m, carry):
        key, idx = carry
        kk = jnp.int32(1) << m

        def row_stage(t, carry):
            key, idx = carry
            jr = (kk >> 8) >> t
            a_k, b_k = row_partner(key, jr, kscr)
            a_i, b_i = row_partner(idx, jr, iscr)
            sel = (row & jr) == 0
            pk = jnp.where(sel, a_k, b_k)
            pi = jnp.where(sel, a_i, b_i)
            return cmpx(key, idx, pk, pi, jr << 7, kk)

        key, idx = lax.fori_loop(0, jnp.maximum(m - 7, 0), row_stage,
                                 (key, idx))
        for s in (64, 32, 16, 8, 4, 2, 1):
            a_k = pltpu.roll(key, LANES - s, axis=1)
            b_k = pltpu.roll(key, s, axis=1)
            a_i = pltpu.roll(idx, LANES - s, axis=1)
            b_i = pltpu.roll(idx, s, axis=1)
            sel = (lane & s) == 0
            pk = jnp.where(sel, a_k, b_k)
            pi = jnp.where(sel, a_i, b_i)
            active = jnp.broadcast_to(jnp.int32(s) <= (kk >> 1), key.shape)
            key, idx = cmpx(key, idx, pk, pi, s, kk, active)
        return key, idx

    v = val_ref[...]
    b = lax.bitcast_convert_type(v, jnp.uint32)
    mask = jnp.where(b >= jnp.uint32(0x80000000), jnp.uint32(0xFFFFFFFF),
                     jnp.uint32(0x80000000))
    u = b ^ mask
    pad = i >= E
    key0 = jnp.where(pad, jnp.uint32(0xFFFFFFFF), u)
    idx0 = jnp.where(pad, jnp.int32(0x7FFFFFFF), i)
    key_s, idx_s = lax.fori_loop(1, 20, phase, (key0, idx0))
    oidx_ref[...] = idx_s

    def row_shift_down(x, q, fill):
        iscr[pl.ds(PADR, R), :] = x
        y = iscr[pl.ds(PADR - q, R), :]
        return jnp.where(row >= q, y, fill)

    def row_shift_up(x, q, fill):
        iscr[pl.ds(PADR, R), :] = x
        y = iscr[pl.ds(PADR + q, R), :]
        return jnp.where(row < R - q, y, fill)

    kscr[pl.ds(PADR, R), :] = key_s
    km1 = kscr[pl.ds(PADR - 1, R), :]
    prev = jnp.where(lane >= 1, pltpu.roll(key_s, 1, axis=1),
                     pltpu.roll(km1, 1, axis=1))
    bdry = (key_s != prev) | (i == 0)
    kp1 = kscr[pl.ds(PADR + 1, R), :]
    nxt = jnp.where(lane < LANES - 1, pltpu.roll(key_s, LANES - 1, axis=1),
                    pltpu.roll(kp1, LANES - 1, axis=1))
    endb = (key_s != nxt) | (i == n - 1)

    p = jnp.where(bdry, i, 0)
    for k in (1, 2, 4, 8, 16, 32, 64):
        sh = pltpu.roll(p, k, axis=1)
        p = jnp.maximum(p, jnp.where(lane >= k, sh, 0))
    t_row = jnp.broadcast_to(jnp.max(p, axis=1, keepdims=True), p.shape)
    x = row_shift_down(t_row, 1, 0)

    def scan_fwd(t, x):
        q = jnp.int32(1) << t
        return jnp.maximum(x, row_shift_down(x, q, 0))

    x = lax.fori_loop(0, 12, scan_fwd, x)
    s_run = jnp.maximum(p, x)

    big = jnp.int32(0x7FFFFFFF)
    p2 = jnp.where(endb, i, big)
    for k in (1, 2, 4, 8, 16, 32, 64):
        sh = pltpu.roll(p2, LANES - k, axis=1)
        p2 = jnp.minimum(p2, jnp.where(lane < LANES - k, sh, big))
    u_row = jnp.broadcast_to(jnp.min(p2, axis=1, keepdims=True), p2.shape)
    y = row_shift_up(u_row, 1, big)

    def scan_bwd(t, y):
        q = jnp.int32(1) << t
        return jnp.minimum(y, row_shift_up(y, q, big))

    y = lax.fori_loop(0, 12, scan_bwd, y)
    e_run = jnp.minimum(p2, y)
    otgt_ref[...] = s_run + e_run - i


_sort_call = pl.pallas_call(
    _sort_kernel,
    in_specs=[
        pl.BlockSpec(memory_space=pltpu.VMEM),
    ],
    out_specs=[
        pl.BlockSpec(memory_space=pltpu.VMEM),
        pl.BlockSpec(memory_space=pltpu.VMEM),
    ],
    out_shape=[
        jax.ShapeDtypeStruct((R, LANES), jnp.int32),
        jax.ShapeDtypeStruct((R, LANES), jnp.int32),
    ],
    scratch_shapes=[
        pltpu.VMEM((R + 2 * PADR, LANES), jnp.uint32),
        pltpu.VMEM((R + 2 * PADR, LANES), jnp.int32),
    ],
)


EFULL = E + N
_NW = 32
_CH = 120
_BPAD = 330240
_EPAD = 322560


def _make_row_gather(BPAD):
    D = 128
    mesh = plsc.VectorSubcoreMesh(core_axis_name="c", subcore_axis_name="s")
    cpw = (BPAD // _NW) // _CH

    @functools.partial(
        pl.kernel,
        mesh=mesh,
        out_type=jax.ShapeDtypeStruct((BPAD, D), jnp.float32),
        scratch_types=[
            pltpu.VMEM((_CH,), jnp.int32),
            pltpu.VMEM((_CH, D), jnp.float32),
            pltpu.SemaphoreType.DMA,
        ],
    )
    def row_gather(table_hbm, idx_hbm, out_hbm, iv, rv, sem):
        wid = lax.axis_index("s") * 2 + lax.axis_index("c")

        def chunk(c, carry):
            off = wid * (cpw * _CH) + c * _CH
            pltpu.sync_copy(idx_hbm.at[pl.ds(off, _CH)], iv)
            pltpu.async_copy(table_hbm.at[iv], rv, sem).wait()
            pltpu.sync_copy(rv, out_hbm.at[pl.ds(off, _CH)])
            return carry

        lax.fori_loop(0, cpw, chunk, 0)

    return row_gather


def _make_conv_gather():
    D = 128
    mesh = plsc.VectorSubcoreMesh(core_axis_name="c", subcore_axis_name="s")
    cpw = (_BPAD // _NW) // _CH

    @functools.partial(
        pl.kernel,
        mesh=mesh,
        out_type=[
            jax.ShapeDtypeStruct((_BPAD, D), jnp.float32),
            jax.ShapeDtypeStruct((_BPAD,), jnp.float32),
            jax.ShapeDtypeStruct((_BPAD,), jnp.float32),
        ],
        scratch_types=[
            pltpu.VMEM((_CH,), jnp.int32),
            pltpu.VMEM((_CH,), jnp.int32),
            pltpu.VMEM((_CH, D), jnp.float32),
            pltpu.VMEM((_CH,), jnp.float32),
            pltpu.VMEM((_CH,), jnp.float32),
            pltpu.SemaphoreType.DMA,
        ],
    )
    def conv_gather(table_hbm, dinv_hbm, src_hbm, dst_hbm,
                    orows, ods, odd, iv, iv2, rv, dv, dv2, sem):
        wid = lax.axis_index("s") * 2 + lax.axis_index("c")

        def chunk(c, carry):
            off = wid * (cpw * _CH) + c * _CH
            pltpu.sync_copy(src_hbm.at[pl.ds(off, _CH)], iv)
            pltpu.sync_copy(dst_hbm.at[pl.ds(off, _CH)], iv2)
            pltpu.async_copy(table_hbm.at[iv], rv, sem).wait()
            pltpu.async_copy(dinv_hbm.at[iv], dv, sem).wait()
            pltpu.async_copy(dinv_hbm.at[iv2], dv2, sem).wait()
            pltpu.sync_copy(rv, orows.at[pl.ds(off, _CH)])
            pltpu.sync_copy(dv, ods.at[pl.ds(off, _CH)])
            pltpu.sync_copy(dv2, odd.at[pl.ds(off, _CH)])
            return carry

        lax.fori_loop(0, cpw, chunk, 0)

    return conv_gather


_conv_gather = _make_conv_gather()


def _gcn_conv(x, edge_index, W, b, gather_fn):
    h = x @ W
    loop = jnp.arange(N, dtype=edge_index.dtype)
    src = jnp.concatenate([edge_index[0], loop])
    dst = jnp.concatenate([edge_index[1], loop])
    deg = jnp.zeros((N,), dtype=h.dtype).at[dst].add(jnp.ones(src.shape[0], dtype=h.dtype))
    dinv = jnp.where(deg > 0, 1.0 / jnp.sqrt(deg), 0.0)
    zpad = jnp.zeros((_BPAD - EFULL,), dtype=src.dtype)
    src_pad = jnp.concatenate([src, zpad])
    dst_pad = jnp.concatenate([dst, zpad])
    h128 = jnp.concatenate(
        [h, jnp.zeros((N, 128 - h.shape[1]), dtype=h.dtype)], axis=1)
    rows, ds, dd = _conv_gather(h128, dinv, src_pad, dst_pad)
    hsrc = rows[:EFULL, :h.shape[1]]
    norm = ds[:EFULL] * dd[:EFULL]
    msg = hsrc * norm[:, None]
    out = jnp.zeros((N, h.shape[1]), dtype=h.dtype).at[dst].add(msg)
    return out + b


_gather_full = _make_row_gather(_BPAD)
_gather_edge = _make_row_gather(_EPAD)


def kernel(x, edge_index, W1, b1, W2, b2):
    xM1 = jax.nn.relu(_gcn_conv(x, edge_index, W1, b1, _gather_full))
    xM2 = _gcn_conv(xM1, edge_index, W2, b2, _gather_full)
    zpad = jnp.zeros((_EPAD - E,), dtype=edge_index.dtype)
    xM2p = jnp.concatenate(
        [xM2, jnp.zeros((N, 128 - OUT_DIM), dtype=xM2.dtype)], axis=1)
    g0 = _gather_edge(xM2p, jnp.concatenate([edge_index[0], zpad]))[:E, :OUT_DIM]
    g1 = _gather_edge(xM2p, jnp.concatenate([edge_index[1], zpad]))[:E, :OUT_DIM]
    value = (g0 * g1).sum(axis=1)
    vp = jnp.zeros((R * LANES,), jnp.float32).at[:E].set(value).reshape(R, LANES)
    oidx, otgt = _sort_call(vp)
    idx_s = oidx.reshape(-1)
    tgt = otgt.reshape(-1)
    het = idx_s[:K_HET]
    homo = jnp.take(idx_s, tgt[E - 1 - jnp.arange(K_HOMO)])
    return (jnp.take(edge_index, homo, axis=1), jnp.take(edge_index, het, axis=1))


# speedup vs baseline: 2.2301x; 1.0237x over previous
import functools

import jax
import jax.numpy as jnp
from jax import lax
from jax.experimental import pallas as pl
from jax.experimental.pallas import tpu as pltpu
from jax.experimental.pallas import tpu_sc as plsc

N = 10000
E = 320000
OUT_DIM = 16
LANES = 128
R = 4096
PADR = 2048
K_HOMO = int(E * 0.8)
K_HET = int(E * 0.2)


def _sort_kernel(val_ref, oidx_ref, otgt_ref, kscr, iscr):
    n = R * LANES
    row = lax.broadcasted_iota(jnp.int32, (R, LANES), 0)
    lane = lax.broadcasted_iota(jnp.int32, (R, LANES), 1)
    i = row * LANES + lane

    def row_partner(x, jr, scr):
        scr[pl.ds(PADR, R), :] = x
        a = scr[pl.ds(PADR + jr, R), :]
        b = scr[pl.ds(PADR - jr, R), :]
        return a, b

    def cmpx(key, idx, pk, pi, jbit, kk, active=None):
        less = (key < pk) | ((key == pk) & (idx < pi))
        take_a = (((i & jbit) == 0) == ((i & kk) == 0)) == less
        if active is not None:
            take_a = take_a | (~active)
        return jnp.where(take_a, key, pk), jnp.where(take_a, idx, pi)

    def phase(m, carry):
        key, idx = carry
        kk = jnp.int32(1) << m

        def row_stage(t, carry):
            key, idx = carry
            jr = (kk >> 8) >> t
            a_k, b_k = row_partner(key, jr, kscr)
            a_i, b_i = row_partner(idx, jr, iscr)
            sel = (row & jr) == 0
            pk = jnp.where(sel, a_k, b_k)
            pi = jnp.where(sel, a_i, b_i)
            return cmpx(key, idx, pk, pi, jr << 7, kk)

        key, idx = lax.fori_loop(0, jnp.maximum(m - 7, 0), row_stage,
                                 (key, idx))
        for s in (64, 32, 16, 8, 4, 2, 1):
            a_k = pltpu.roll(key, LANES - s, axis=1)
            b_k = pltpu.roll(key, s, axis=1)
            a_i = pltpu.roll(idx, LANES - s, axis=1)
            b_i = pltpu.roll(idx, s, axis=1)
            sel = (lane & s) == 0
            pk = jnp.where(sel, a_k, b_k)
            pi = jnp.where(sel, a_i, b_i)
            active = jnp.broadcast_to(jnp.int32(s) <= (kk >> 1), key.shape)
            key, idx = cmpx(key, idx, pk, pi, s, kk, active)
        return key, idx

    v = val_ref[...]
    b = lax.bitcast_convert_type(v, jnp.uint32)
    mask = jnp.where(b >= jnp.uint32(0x80000000), jnp.uint32(0xFFFFFFFF),
                     jnp.uint32(0x80000000))
    u = b ^ mask
    pad = i >= E
    key0 = jnp.where(pad, jnp.uint32(0xFFFFFFFF), u)
    idx0 = jnp.where(pad, jnp.int32(0x7FFFFFFF), i)
    key_s, idx_s = lax.fori_loop(1, 20, phase, (key0, idx0))
    oidx_ref[...] = idx_s

    def row_shift_down(x, q, fill):
        iscr[pl.ds(PADR, R), :] = x
        y = iscr[pl.ds(PADR - q, R), :]
        return jnp.where(row >= q, y, fill)

    def row_shift_up(x, q, fill):
        iscr[pl.ds(PADR, R), :] = x
        y = iscr[pl.ds(PADR + q, R), :]
        return jnp.where(row < R - q, y, fill)

    kscr[pl.ds(PADR, R), :] = key_s
    km1 = kscr[pl.ds(PADR - 1, R), :]
    prev = jnp.where(lane >= 1, pltpu.roll(key_s, 1, axis=1),
                     pltpu.roll(km1, 1, axis=1))
    bdry = (key_s != prev) | (i == 0)
    kp1 = kscr[pl.ds(PADR + 1, R), :]
    nxt = jnp.where(lane < LANES - 1, pltpu.roll(key_s, LANES - 1, axis=1),
                    pltpu.roll(kp1, LANES - 1, axis=1))
    endb = (key_s != nxt) | (i == n - 1)

    p = jnp.where(bdry, i, 0)
    for k in (1, 2, 4, 8, 16, 32, 64):
        sh = pltpu.roll(p, k, axis=1)
        p = jnp.maximum(p, jnp.where(lane >= k, sh, 0))
    t_row = jnp.broadcast_to(jnp.max(p, axis=1, keepdims=True), p.shape)
    x = row_shift_down(t_row, 1, 0)

    def scan_fwd(t, x):
        q = jnp.int32(1) << t
        return jnp.maximum(x, row_shift_down(x, q, 0))

    x = lax.fori_loop(0, 12, scan_fwd, x)
    s_run = jnp.maximum(p, x)

    big = jnp.int32(0x7FFFFFFF)
    p2 = jnp.where(endb, i, big)
    for k in (1, 2, 4, 8, 16, 32, 64):
        sh = pltpu.roll(p2, LANES - k, axis=1)
        p2 = jnp.minimum(p2, jnp.where(lane < LANES - k, sh, big))
    u_row = jnp.broadcast_to(jnp.min(p2, axis=1, keepdims=True), p2.shape)
    y = row_shift_up(u_row, 1, big)

    def scan_bwd(t, y):
        q = jnp.int32(1) << t
        return jnp.minimum(y, row_shift_up(y, q, big))

    y = lax.fori_loop(0, 12, scan_bwd, y)
    e_run = jnp.minimum(p2, y)
    otgt_ref[...] = s_run + e_run - i


_sort_call = pl.pallas_call(
    _sort_kernel,
    in_specs=[
        pl.BlockSpec(memory_space=pltpu.VMEM),
    ],
    out_specs=[
        pl.BlockSpec(memory_space=pltpu.VMEM),
        pl.BlockSpec(memory_space=pltpu.VMEM),
    ],
    out_shape=[
        jax.ShapeDtypeStruct((R, LANES), jnp.int32),
        jax.ShapeDtypeStruct((R, LANES), jnp.int32),
    ],
    scratch_shapes=[
        pltpu.VMEM((R + 2 * PADR, LANES), jnp.uint32),
        pltpu.VMEM((R + 2 * PADR, LANES), jnp.int32),
    ],
)


EFULL = E + N
_NW = 32
_CH = 120
_BPAD = 330240
_EPAD = 322560


def _make_row_gather(BPAD):
    D = 128
    mesh = plsc.VectorSubcoreMesh(core_axis_name="c", subcore_axis_name="s")
    cpw = (BPAD // _NW) // _CH

    @functools.partial(
        pl.kernel,
        mesh=mesh,
        out_type=jax.ShapeDtypeStruct((BPAD, D), jnp.float32),
        scratch_types=[
            pltpu.VMEM((_CH,), jnp.int32),
            pltpu.VMEM((_CH, D), jnp.float32),
            pltpu.SemaphoreType.DMA,
        ],
    )
    def row_gather(table_hbm, idx_hbm, out_hbm, iv, rv, sem):
        wid = lax.axis_index("s") * 2 + lax.axis_index("c")

        def chunk(c, carry):
            off = wid * (cpw * _CH) + c * _CH
            pltpu.sync_copy(idx_hbm.at[pl.ds(off, _CH)], iv)
            pltpu.async_copy(table_hbm.at[iv], rv, sem).wait()
            pltpu.sync_copy(rv, out_hbm.at[pl.ds(off, _CH)])
            return carry

        lax.fori_loop(0, cpw, chunk, 0)

    return row_gather


def _make_conv_gather():
    D = 128
    mesh = plsc.VectorSubcoreMesh(core_axis_name="c", subcore_axis_name="s")
    cpw = (_BPAD // _NW) // _CH

    @functools.partial(
        pl.kernel,
        mesh=mesh,
        out_type=[
            jax.ShapeDtypeStruct((_BPAD, D), jnp.float32),
            jax.ShapeDtypeStruct((_BPAD,), jnp.float32),
            jax.ShapeDtypeStruct((_BPAD,), jnp.float32),
        ],
        scratch_types=[
            pltpu.VMEM((_CH,), jnp.int32),
            pltpu.VMEM((_CH,), jnp.int32),
            pltpu.VMEM((_CH, D), jnp.float32),
            pltpu.VMEM((_CH,), jnp.float32),
            pltpu.VMEM((_CH,), jnp.float32),
            pltpu.SemaphoreType.DMA,
        ],
    )
    def conv_gather(table_hbm, dinv_hbm, src_hbm, dst_hbm,
                    orows, ods, odd, iv, iv2, rv, dv, dv2, sem):
        wid = lax.axis_index("s") * 2 + lax.axis_index("c")

        def chunk(c, carry):
            off = wid * (cpw * _CH) + c * _CH
            pltpu.sync_copy(src_hbm.at[pl.ds(off, _CH)], iv)
            pltpu.sync_copy(dst_hbm.at[pl.ds(off, _CH)], iv2)
            c1 = pltpu.async_copy(table_hbm.at[iv], rv, sem)
            c2 = pltpu.async_copy(dinv_hbm.at[iv], dv, sem)
            c3 = pltpu.async_copy(dinv_hbm.at[iv2], dv2, sem)
            c1.wait()
            c2.wait()
            c3.wait()
            pltpu.sync_copy(rv, orows.at[pl.ds(off, _CH)])
            pltpu.sync_copy(dv, ods.at[pl.ds(off, _CH)])
            pltpu.sync_copy(dv2, odd.at[pl.ds(off, _CH)])
            return carry

        lax.fori_loop(0, cpw, chunk, 0)

    return conv_gather


_conv_gather = _make_conv_gather()


def _gcn_conv(x, edge_index, W, b, gather_fn):
    h = x @ W
    loop = jnp.arange(N, dtype=edge_index.dtype)
    src = jnp.concatenate([edge_index[0], loop])
    dst = jnp.concatenate([edge_index[1], loop])
    deg = jnp.zeros((N,), dtype=h.dtype).at[dst].add(jnp.ones(src.shape[0], dtype=h.dtype))
    dinv = jnp.where(deg > 0, 1.0 / jnp.sqrt(deg), 0.0)
    zpad = jnp.zeros((_BPAD - EFULL,), dtype=src.dtype)
    src_pad = jnp.concatenate([src, zpad])
    dst_pad = jnp.concatenate([dst, zpad])
    h128 = jnp.concatenate(
        [h, jnp.zeros((N, 128 - h.shape[1]), dtype=h.dtype)], axis=1)
    rows, ds, dd = _conv_gather(h128, dinv, src_pad, dst_pad)
    hsrc = rows[:EFULL, :h.shape[1]]
    norm = ds[:EFULL] * dd[:EFULL]
    msg = hsrc * norm[:, None]
    out = jnp.zeros((N, h.shape[1]), dtype=h.dtype).at[dst].add(msg)
    return out + b


_gather_full = _make_row_gather(_BPAD)
_gather_edge = _make_row_gather(_EPAD)


def kernel(x, edge_index, W1, b1, W2, b2):
    xM1 = jax.nn.relu(_gcn_conv(x, edge_index, W1, b1, _gather_full))
    xM2 = _gcn_conv(xM1, edge_index, W2, b2, _gather_full)
    zpad = jnp.zeros((_EPAD - E,), dtype=edge_index.dtype)
    xM2p = jnp.concatenate(
        [xM2, jnp.zeros((N, 128 - OUT_DIM), dtype=xM2.dtype)], axis=1)
    g0 = _gather_edge(xM2p, jnp.concatenate([edge_index[0], zpad]))[:E, :OUT_DIM]
    g1 = _gather_edge(xM2p, jnp.concatenate([edge_index[1], zpad]))[:E, :OUT_DIM]
    value = (g0 * g1).sum(axis=1)
    vp = jnp.zeros((R * LANES,), jnp.float32).at[:E].set(value).reshape(R, LANES)
    oidx, otgt = _sort_call(vp)
    idx_s = oidx.reshape(-1)
    tgt = otgt.reshape(-1)
    het = idx_s[:K_HET]
    homo = jnp.take(idx_s, tgt[E - 1 - jnp.arange(K_HOMO)])
    return (jnp.take(edge_index, homo, axis=1), jnp.take(edge_index, het, axis=1))


# speedup vs baseline: 2.2303x; 1.0001x over previous
import functools

import jax
import jax.numpy as jnp
from jax import lax
from jax.experimental import pallas as pl
from jax.experimental.pallas import tpu as pltpu
from jax.experimental.pallas import tpu_sc as plsc

N = 10000
E = 320000
OUT_DIM = 16
LANES = 128
R = 4096
PADR = 2048
K_HOMO = int(E * 0.8)
K_HET = int(E * 0.2)


def _sort_kernel(val_ref, oidx_ref, otgt_ref, kscr, iscr):
    n = R * LANES
    row = lax.broadcasted_iota(jnp.int32, (R, LANES), 0)
    lane = lax.broadcasted_iota(jnp.int32, (R, LANES), 1)
    i = row * LANES + lane

    def row_partner(x, jr, scr):
        scr[pl.ds(PADR, R), :] = x
        a = scr[pl.ds(PADR + jr, R), :]
        b = scr[pl.ds(PADR - jr, R), :]
        return a, b

    def cmpx(key, idx, pk, pi, jbit, kk, active=None):
        less = (key < pk) | ((key == pk) & (idx < pi))
        take_a = (((i & jbit) == 0) == ((i & kk) == 0)) == less
        if active is not None:
            take_a = take_a | (~active)
        return jnp.where(take_a, key, pk), jnp.where(take_a, idx, pi)

    def phase(m, carry):
        key, idx = carry
        kk = jnp.int32(1) << m

        def row_stage(t, carry):
            key, idx = carry
            jr = (kk >> 8) >> t
            a_k, b_k = row_partner(key, jr, kscr)
            a_i, b_i = row_partner(idx, jr, iscr)
            sel = (row & jr) == 0
            pk = jnp.where(sel, a_k, b_k)
            pi = jnp.where(sel, a_i, b_i)
            return cmpx(key, idx, pk, pi, jr << 7, kk)

        key, idx = lax.fori_loop(0, jnp.maximum(m - 7, 0), row_stage,
                                 (key, idx))
        for s in (64, 32, 16, 8, 4, 2, 1):
            a_k = pltpu.roll(key, LANES - s, axis=1)
            b_k = pltpu.roll(key, s, axis=1)
            a_i = pltpu.roll(idx, LANES - s, axis=1)
            b_i = pltpu.roll(idx, s, axis=1)
            sel = (lane & s) == 0
            pk = jnp.where(sel, a_k, b_k)
            pi = jnp.where(sel, a_i, b_i)
            active = jnp.broadcast_to(jnp.int32(s) <= (kk >> 1), key.shape)
            key, idx = cmpx(key, idx, pk, pi, s, kk, active)
        return key, idx

    v = val_ref[...]
    b = lax.bitcast_convert_type(v, jnp.uint32)
    mask = jnp.where(b >= jnp.uint32(0x80000000), jnp.uint32(0xFFFFFFFF),
                     jnp.uint32(0x80000000))
    u = b ^ mask
    pad = i >= E
    key0 = jnp.where(pad, jnp.uint32(0xFFFFFFFF), u)
    idx0 = jnp.where(pad, jnp.int32(0x7FFFFFFF), i)
    key_s, idx_s = lax.fori_loop(1, 20, phase, (key0, idx0))
    oidx_ref[...] = idx_s

    def row_shift_down(x, q, fill):
        iscr[pl.ds(PADR, R), :] = x
        y = iscr[pl.ds(PADR - q, R), :]
        return jnp.where(row >= q, y, fill)

    def row_shift_up(x, q, fill):
        iscr[pl.ds(PADR, R), :] = x
        y = iscr[pl.ds(PADR + q, R), :]
        return jnp.where(row < R - q, y, fill)

    kscr[pl.ds(PADR, R), :] = key_s
    km1 = kscr[pl.ds(PADR - 1, R), :]
    prev = jnp.where(lane >= 1, pltpu.roll(key_s, 1, axis=1),
                     pltpu.roll(km1, 1, axis=1))
    bdry = (key_s != prev) | (i == 0)
    kp1 = kscr[pl.ds(PADR + 1, R), :]
    nxt = jnp.where(lane < LANES - 1, pltpu.roll(key_s, LANES - 1, axis=1),
                    pltpu.roll(kp1, LANES - 1, axis=1))
    endb = (key_s != nxt) | (i == n - 1)

    p = jnp.where(bdry, i, 0)
    for k in (1, 2, 4, 8, 16, 32, 64):
        sh = pltpu.roll(p, k, axis=1)
        p = jnp.maximum(p, jnp.where(lane >= k, sh, 0))
    t_row = jnp.broadcast_to(jnp.max(p, axis=1, keepdims=True), p.shape)
    x = row_shift_down(t_row, 1, 0)

    def scan_fwd(t, x):
        q = jnp.int32(1) << t
        return jnp.maximum(x, row_shift_down(x, q, 0))

    x = lax.fori_loop(0, 12, scan_fwd, x)
    s_run = jnp.maximum(p, x)

    big = jnp.int32(0x7FFFFFFF)
    p2 = jnp.where(endb, i, big)
    for k in (1, 2, 4, 8, 16, 32, 64):
        sh = pltpu.roll(p2, LANES - k, axis=1)
        p2 = jnp.minimum(p2, jnp.where(lane < LANES - k, sh, big))
    u_row = jnp.broadcast_to(jnp.min(p2, axis=1, keepdims=True), p2.shape)
    y = row_shift_up(u_row, 1, big)

    def scan_bwd(t, y):
        q = jnp.int32(1) << t
        return jnp.minimum(y, row_shift_up(y, q, big))

    y = lax.fori_loop(0, 12, scan_bwd, y)
    e_run = jnp.minimum(p2, y)
    otgt_ref[...] = s_run + e_run - i


_sort_call = pl.pallas_call(
    _sort_kernel,
    in_specs=[
        pl.BlockSpec(memory_space=pltpu.VMEM),
    ],
    out_specs=[
        pl.BlockSpec(memory_space=pltpu.VMEM),
        pl.BlockSpec(memory_space=pltpu.VMEM),
    ],
    out_shape=[
        jax.ShapeDtypeStruct((R, LANES), jnp.int32),
        jax.ShapeDtypeStruct((R, LANES), jnp.int32),
    ],
    scratch_shapes=[
        pltpu.VMEM((R + 2 * PADR, LANES), jnp.uint32),
        pltpu.VMEM((R + 2 * PADR, LANES), jnp.int32),
    ],
)


EFULL = E + N
_NW = 32
_CH = 120
_BPAD = 330240
_EPAD = 322560


def _make_row_gather(BPAD):
    D = 128
    mesh = plsc.VectorSubcoreMesh(core_axis_name="c", subcore_axis_name="s")
    cpw = (BPAD // _NW) // _CH

    @functools.partial(
        pl.kernel,
        mesh=mesh,
        out_type=jax.ShapeDtypeStruct((BPAD, D), jnp.float32),
        scratch_types=[
            pltpu.VMEM((_CH,), jnp.int32),
            pltpu.VMEM((_CH, D), jnp.float32),
            pltpu.SemaphoreType.DMA,
        ],
    )
    def row_gather(table_hbm, idx_hbm, out_hbm, iv, rv, sem):
        wid = lax.axis_index("s") * 2 + lax.axis_index("c")

        def chunk(c, carry):
            off = wid * (cpw * _CH) + c * _CH
            pltpu.sync_copy(idx_hbm.at[pl.ds(off, _CH)], iv)
            pltpu.async_copy(table_hbm.at[iv], rv, sem).wait()
            pltpu.sync_copy(rv, out_hbm.at[pl.ds(off, _CH)])
            return carry

        lax.fori_loop(0, cpw, chunk, 0)

    return row_gather


def _make_conv_gather():
    D = 128
    mesh = plsc.VectorSubcoreMesh(core_axis_name="c", subcore_axis_name="s")
    cpw = (_BPAD // _NW) // _CH

    @functools.partial(
        pl.kernel,
        mesh=mesh,
        out_type=[
            jax.ShapeDtypeStruct((_BPAD, D), jnp.float32),
            jax.ShapeDtypeStruct((_BPAD,), jnp.float32),
            jax.ShapeDtypeStruct((_BPAD,), jnp.float32),
        ],
        scratch_types=[
            pltpu.VMEM((_CH,), jnp.int32),
            pltpu.VMEM((_CH,), jnp.int32),
            pltpu.VMEM((_CH, D), jnp.float32),
            pltpu.VMEM((_CH,), jnp.float32),
            pltpu.VMEM((_CH,), jnp.float32),
            pltpu.SemaphoreType.DMA,
        ],
    )
    def conv_gather(table_hbm, dinv_hbm, src_hbm, dst_hbm,
                    orows, ods, odd, iv, iv2, rv, dv, dv2, sem):
        wid = lax.axis_index("s") * 2 + lax.axis_index("c")

        def chunk(c, carry):
            off = wid * (cpw * _CH) + c * _CH
            pltpu.sync_copy(src_hbm.at[pl.ds(off, _CH)], iv)
            pltpu.sync_copy(dst_hbm.at[pl.ds(off, _CH)], iv2)
            c1 = pltpu.async_copy(table_hbm.at[iv], rv, sem)
            c2 = pltpu.async_copy(dinv_hbm.at[iv], dv, sem)
            c3 = pltpu.async_copy(dinv_hbm.at[iv2], dv2, sem)
            c1.wait()
            c2.wait()
            c3.wait()
            pltpu.sync_copy(rv, orows.at[pl.ds(off, _CH)])
            pltpu.sync_copy(dv, ods.at[pl.ds(off, _CH)])
            pltpu.sync_copy(dv2, odd.at[pl.ds(off, _CH)])
            return carry

        lax.fori_loop(0, cpw, chunk, 0)

    return conv_gather


_conv_gather = _make_conv_gather()


def _gcn_conv(x, edge_index, W, b):
    h = x @ W
    loop = jnp.arange(N, dtype=edge_index.dtype)
    src = jnp.concatenate([edge_index[0], loop])
    dst = jnp.concatenate([edge_index[1], loop])
    deg = jnp.zeros((N,), dtype=h.dtype).at[dst].add(jnp.ones(src.shape[0], dtype=h.dtype))
    dinv = jnp.where(deg > 0, 1.0 / jnp.sqrt(deg), 0.0)
    zpad = jnp.zeros((_BPAD - EFULL,), dtype=src.dtype)
    src_pad = jnp.concatenate([src, zpad])
    dst_pad = jnp.concatenate([dst, zpad])
    h128 = jnp.concatenate(
        [h, jnp.zeros((N, 128 - h.shape[1]), dtype=h.dtype)], axis=1)
    rows, ds, dd = _conv_gather(h128, dinv, src_pad, dst_pad)
    hsrc = rows[:EFULL, :h.shape[1]]
    norm = ds[:EFULL] * dd[:EFULL]
    msg = hsrc * norm[:, None]
    out = jnp.zeros((N, h.shape[1]), dtype=h.dtype).at[dst].add(msg)
    return out + b


_gather_edge = _make_row_gather(_EPAD)


def kernel(x, edge_index, W1, b1, W2, b2):
    xM1 = jax.nn.relu(_gcn_conv(x, edge_index, W1, b1))
    xM2 = _gcn_conv(xM1, edge_index, W2, b2)
    zpad = jnp.zeros((_EPAD - E,), dtype=edge_index.dtype)
    xM2p = jnp.concatenate(
        [xM2, jnp.zeros((N, 128 - OUT_DIM), dtype=xM2.dtype)], axis=1)
    g0 = _gather_edge(xM2p, jnp.concatenate([edge_index[0], zpad]))[:E, :OUT_DIM]
    g1 = _gather_edge(xM2p, jnp.concatenate([edge_index[1], zpad]))[:E, :OUT_DIM]
    value = (g0 * g1).sum(axis=1)
    vp = jnp.zeros((R * LANES,), jnp.float32).at[:E].set(value).reshape(R, LANES)
    oidx, otgt = _sort_call(vp)
    idx_s = oidx.reshape(-1)
    tgt = otgt.reshape(-1)
    het = idx_s[:K_HET]
    homo = jnp.take(idx_s, tgt[E - 1 - jnp.arange(K_HOMO)])
    return (jnp.take(edge_index, homo, axis=1), jnp.take(edge_index, het, axis=1))
